# baseline (speedup 1.0000x reference)
import functools

import jax
import jax.numpy as jnp
from jax import lax
from jax.experimental import pallas as pl
from jax.experimental.pallas import tpu as pltpu
from jax.experimental.pallas import tpu_sc as plsc


def _router_body(x_ref, w1_ref, b1_ref, w3_ref, b3_ref, idx_ref):
    x = x_ref[...]
    h = jnp.maximum(
        jnp.dot(x, w1_ref[...], preferred_element_type=jnp.float32) + b1_ref[...],
        0.0,
    )
    logits = jnp.dot(h, w3_ref[...], preferred_element_type=jnp.float32) + b3_ref[...]
    idx_ref[0, 0, :] = jnp.argmax(logits, axis=-1).astype(jnp.int32)


def _meta_body(idxq_ref, idxd_ref, dest_ref, bexp_ref, *, E, T, NBLK, R, C):
    idx = jnp.concatenate([idxq_ref[...], idxd_ref[...]], axis=0)
    ohs = [idx == e for e in range(E)]
    counts = [jnp.sum(oh.astype(jnp.int32)) for oh in ohs]

    starts = []
    s = jnp.int32(0)
    for e in range(E):
        starts.append(s)
        s = s + ((counts[e] + (T - 1)) // T) * T

    cio = lax.broadcasted_iota(jnp.int32, (C, C), 0)
    cjo = lax.broadcasted_iota(jnp.int32, (C, C), 1)
    su = (cio < cjo).astype(jnp.float32)
    rio = lax.broadcasted_iota(jnp.int32, (R, R), 0)
    rjo = lax.broadcasted_iota(jnp.int32, (R, R), 1)
    sl = (rjo < rio).astype(jnp.float32)

    dest = jnp.zeros((R, C), jnp.int32)
    for e in range(E):
        ohf = ohs[e].astype(jnp.float32)
        within = jnp.dot(ohf, su, preferred_element_type=jnp.float32)
        rowsum = jnp.sum(ohf, axis=1, keepdims=True)
        rexcl = jnp.dot(sl, rowsum, preferred_element_type=jnp.float32)
        pos = (rexcl + within).astype(jnp.int32)
        dest = dest + jnp.where(ohs[e], starts[e] + pos, 0)
    dest_ref[...] = dest

    bio = lax.broadcasted_iota(jnp.int32, (1, NBLK), 1) * T
    be = jnp.zeros((1, NBLK), jnp.int32)
    for e in range(1, E):
        be = be + (bio >= starts[e]).astype(jnp.int32)
    bexp_ref[...] = be


def _expert_body(bexp_ref, x0_ref, x1_ref, x2_ref, x3_ref,
                 w1_ref, b1_ref, w2_ref, b2_ref, o_ref, *, DS):
    xs = [x0_ref[...], x1_ref[...], x2_ref[...], x3_ref[...]]
    h = jnp.dot(xs[0].astype(jnp.bfloat16), w1_ref[0, 0],
                preferred_element_type=jnp.float32)
    for k in range(1, 4):
        h = h + jnp.dot(xs[k].astype(jnp.bfloat16), w1_ref[0, k],
                        preferred_element_type=jnp.float32)
    h = jnp.maximum(h + b1_ref[0], 0.0)
    o = jnp.dot(h.astype(jnp.bfloat16), w2_ref[0],
                preferred_element_type=jnp.float32) + b2_ref[0]
    n = jnp.sqrt(jnp.sum(o * o, axis=1, keepdims=True))
    y = o / jnp.maximum(n, 1e-6)
    for k in range(4):
        o_ref[:, k * DS:(k + 1) * DS] = y[:, k * DS:(k + 1) * DS] + xs[k]


def kernel(query_emb, doc_emb, W1, b1, W3, b3, Ws1, bs1, Ws2, bs2):
    B, D = query_emb.shape
    H = W1.shape[1]
    E = W3.shape[1]
    N = 2 * B
    T = 256
    NBLK = N // T + E
    NPAD = NBLK * T
    f32 = jnp.float32

    TB = 512
    nb = B // TB

    def run_router(x):
        return pl.pallas_call(
            _router_body,
            grid=(nb,),
            in_specs=[
                pl.BlockSpec((TB, D), lambda i: (i, 0)),
                pl.BlockSpec((D, H), lambda i: (0, 0)),
                pl.BlockSpec((1, H), lambda i: (0, 0)),
                pl.BlockSpec((H, E), lambda i: (0, 0)),
                pl.BlockSpec((1, E), lambda i: (0, 0)),
            ],
            out_specs=pl.BlockSpec((1, 1, TB), lambda i: (i, 0, 0)),
            out_shape=jax.ShapeDtypeStruct((nb, 1, TB), jnp.int32),
        )(x, W1, b1.reshape(1, H), W3, b3.reshape(1, E))

    idxq = run_router(query_emb)
    idxd = run_router(doc_emb)

    C = 128
    R = N // C
    dest2d, bexp2d = pl.pallas_call(
        functools.partial(_meta_body, E=E, T=T, NBLK=NBLK, R=R, C=C),
        grid=(1,),
        in_specs=[
            pl.BlockSpec((R // 2, C), lambda i: (0, 0)),
            pl.BlockSpec((R // 2, C), lambda i: (0, 0)),
        ],
        out_specs=[
            pl.BlockSpec((R, C), lambda i: (0, 0)),
            pl.BlockSpec((1, NBLK), lambda i: (0, 0)),
        ],
        out_shape=[
            jax.ShapeDtypeStruct((R, C), jnp.int32),
            jax.ShapeDtypeStruct((1, NBLK), jnp.int32),
        ],
    )(idxq.reshape(R // 2, C), idxd.reshape(R // 2, C))
    dest = dest2d.reshape(N)
    bexp = bexp2d.reshape(NBLK)
    destq = dest[:B].reshape(1, B)
    destd = dest[B:].reshape(1, B)

    S = 4
    DS = D // S
    mesh = plsc.VectorSubcoreMesh(core_axis_name="c", subcore_axis_name="s")
    W = 128

    @functools.partial(
        pl.kernel,
        mesh=mesh,
        out_type=[jax.ShapeDtypeStruct((NPAD, DS), f32) for _ in range(S)],
    )
    def sc_scatter(q_hbm, d_hbm, iq_hbm, id_hbm, xo0, xo1, xo2, xo3):
        outs = (xo0, xo1, xo2, xo3)
        for k in range(S):
            for src, idx in ((q_hbm, iq_hbm), (d_hbm, id_hbm)):
                def body(x_vmem, i_vmem, _out=outs[k]):
                    pltpu.sync_copy(x_vmem, _out.at[i_vmem.at[0]])

                pltpu.emit_pipeline(
                    body,
                    grid=(B // W,),
                    in_specs=[
                        pl.BlockSpec((W, DS), lambda i, _k=k: (i, _k)),
                        pl.BlockSpec((1, W), lambda i: (0, i)),
                    ],
                    out_specs=[],
                    core_axis_name=("c", "s"),
                    dimension_semantics=(pltpu.PARALLEL,),
                )(src, idx)

    xs_planes = sc_scatter(query_emb, doc_emb, destq, destd)

    grid_spec = pltpu.PrefetchScalarGridSpec(
        num_scalar_prefetch=1,
        grid=(NBLK,),
        in_specs=[
            pl.BlockSpec((T, DS), lambda i, be: (i, 0)),
            pl.BlockSpec((T, DS), lambda i, be: (i, 0)),
            pl.BlockSpec((T, DS), lambda i, be: (i, 0)),
            pl.BlockSpec((T, DS), lambda i, be: (i, 0)),
            pl.BlockSpec((1, S, DS, H), lambda i, be: (be[i], 0, 0, 0)),
            pl.BlockSpec((1, 1, H), lambda i, be: (be[i], 0, 0)),
            pl.BlockSpec((1, H, D), lambda i, be: (be[i], 0, 0)),
            pl.BlockSpec((1, 1, D), lambda i, be: (be[i], 0, 0)),
        ],
        out_specs=pl.BlockSpec((T, D), lambda i, be: (i, 0)),
    )
    res = pl.pallas_call(
        functools.partial(_expert_body, DS=DS),
        grid_spec=grid_spec,
        out_shape=jax.ShapeDtypeStruct((NPAD, D), f32),
    )(bexp, *xs_planes, Ws1.reshape(E, S, DS, H).astype(jnp.bfloat16),
      bs1.reshape(E, 1, H), Ws2.astype(jnp.bfloat16), bs2.reshape(E, 1, D))

    NW = 32
    RPW = N // NW
    CH = 32
    NCH = RPW // CH

    @functools.partial(
        pl.kernel,
        mesh=mesh,
        out_type=[
            jax.ShapeDtypeStruct((B, D), f32),
            jax.ShapeDtypeStruct((B, D), f32),
        ],
        scratch_types=[
            pltpu.VMEM((CH,), jnp.int32),
            pltpu.VMEM((CH,), jnp.int32),
            pltpu.VMEM((CH, D), f32),
            pltpu.VMEM((CH, D), f32),
            pltpu.SemaphoreType.DMA,
            pltpu.SemaphoreType.DMA,
        ],
    )
    def sc_gather(res_hbm, i_hbm, q_hbm, d_hbm,
                  idx_v0, idx_v1, rows_v0, rows_v1, sem0, sem1):
        wid = lax.axis_index("s") * 2 + lax.axis_index("c")
        base = wid * RPW
        idx_v = (idx_v0, idx_v1)
        rows_v = (rows_v0, rows_v1)
        sems = (sem0, sem1)

        pltpu.sync_copy(i_hbm.at[pl.ds(base, CH)], idx_v[0])
        pltpu.make_async_copy(res_hbm.at[idx_v[0]], rows_v[0], sems[0]).start()
        for j in range(NCH):
            cur = j % 2
            nxt = 1 - cur
            if j + 1 < NCH:
                pltpu.sync_copy(
                    i_hbm.at[pl.ds(base + (j + 1) * CH, CH)], idx_v[nxt])
                pltpu.make_async_copy(
                    res_hbm.at[idx_v[nxt]], rows_v[nxt], sems[nxt]).start()
            pltpu.make_async_copy(
                res_hbm.at[idx_v[cur]], rows_v[cur], sems[cur]).wait()
            row0 = base + j * CH

            @pl.when(wid < NW // 2)
            def _():
                pltpu.sync_copy(rows_v[cur], q_hbm.at[pl.ds(row0, CH)])

            @pl.when(wid >= NW // 2)
            def _():
                pltpu.sync_copy(rows_v[cur], d_hbm.at[pl.ds(row0 - B, CH)])

    q_out, d_out = sc_gather(res, dest)
    return (q_out, d_out)


# speedup vs baseline: 1.0285x; 1.0285x over previous
import functools

import jax
import jax.numpy as jnp
from jax import lax
from jax.experimental import pallas as pl
from jax.experimental.pallas import tpu as pltpu
from jax.experimental.pallas import tpu_sc as plsc


def _router_body(x_ref, w1_ref, b1_ref, w3_ref, b3_ref, idx_ref):
    x = x_ref[...]
    h = jnp.maximum(
        jnp.dot(x, w1_ref[...], preferred_element_type=jnp.float32) + b1_ref[...],
        0.0,
    )
    logits = jnp.dot(h, w3_ref[...], preferred_element_type=jnp.float32) + b3_ref[...]
    idx_ref[0, 0, :] = jnp.argmax(logits, axis=-1).astype(jnp.int32)


def _meta_body(idxq_ref, idxd_ref, dest_ref, bexp_ref, *, E, T, NBLK, R, C):
    idx = jnp.concatenate([idxq_ref[...], idxd_ref[...]], axis=0)
    ohs = [idx == e for e in range(E)]
    counts = [jnp.sum(oh.astype(jnp.int32)) for oh in ohs]

    starts = []
    s = jnp.int32(0)
    for e in range(E):
        starts.append(s)
        s = s + ((counts[e] + (T - 1)) // T) * T

    cio = lax.broadcasted_iota(jnp.int32, (C, C), 0)
    cjo = lax.broadcasted_iota(jnp.int32, (C, C), 1)
    su = (cio < cjo).astype(jnp.float32)
    rio = lax.broadcasted_iota(jnp.int32, (R, R), 0)
    rjo = lax.broadcasted_iota(jnp.int32, (R, R), 1)
    sl = (rjo < rio).astype(jnp.float32)

    dest = jnp.zeros((R, C), jnp.int32)
    for e in range(E):
        ohf = ohs[e].astype(jnp.float32)
        within = jnp.dot(ohf, su, preferred_element_type=jnp.float32)
        rowsum = jnp.sum(ohf, axis=1, keepdims=True)
        rexcl = jnp.dot(sl, rowsum, preferred_element_type=jnp.float32)
        pos = (rexcl + within).astype(jnp.int32)
        dest = dest + jnp.where(ohs[e], starts[e] + pos, 0)
    dest_ref[...] = dest

    bio = lax.broadcasted_iota(jnp.int32, (1, NBLK), 1) * T
    be = jnp.zeros((1, NBLK), jnp.int32)
    for e in range(1, E):
        be = be + (bio >= starts[e]).astype(jnp.int32)
    bexp_ref[...] = be


def _expert_body(bexp_ref, x0_ref, x1_ref, x2_ref, x3_ref,
                 w1_ref, b1_ref, w2_ref, b2_ref, o_ref,
                 w1s_ref, w2s_ref, *, DS):
    i = pl.program_id(0)
    prev = bexp_ref[jnp.maximum(i - 1, 0)]

    @pl.when(jnp.logical_or(i == 0, bexp_ref[i] != prev))
    def _():
        w1s_ref[...] = w1_ref[0].astype(jnp.bfloat16)
        w2s_ref[...] = w2_ref[0].astype(jnp.bfloat16)

    xs = [x0_ref[...], x1_ref[...], x2_ref[...], x3_ref[...]]
    h = jnp.dot(xs[0].astype(jnp.bfloat16), w1s_ref[0],
                preferred_element_type=jnp.float32)
    for k in range(1, 4):
        h = h + jnp.dot(xs[k].astype(jnp.bfloat16), w1s_ref[k],
                        preferred_element_type=jnp.float32)
    h = jnp.maximum(h + b1_ref[0], 0.0)
    o = jnp.dot(h.astype(jnp.bfloat16), w2s_ref[...],
                preferred_element_type=jnp.float32) + b2_ref[0]
    inv = 1.0 / jnp.maximum(jnp.sqrt(jnp.sum(o * o, axis=1, keepdims=True)),
                            1e-6)
    y = o * inv
    for k in range(4):
        o_ref[:, k * DS:(k + 1) * DS] = y[:, k * DS:(k + 1) * DS] + xs[k]


def kernel(query_emb, doc_emb, W1, b1, W3, b3, Ws1, bs1, Ws2, bs2):
    B, D = query_emb.shape
    H = W1.shape[1]
    E = W3.shape[1]
    N = 2 * B
    T = 256
    NBLK = N // T + E
    NPAD = NBLK * T
    f32 = jnp.float32

    TB = 512
    nb = B // TB

    def run_router(x):
        return pl.pallas_call(
            _router_body,
            grid=(nb,),
            in_specs=[
                pl.BlockSpec((TB, D), lambda i: (i, 0)),
                pl.BlockSpec((D, H), lambda i: (0, 0)),
                pl.BlockSpec((1, H), lambda i: (0, 0)),
                pl.BlockSpec((H, E), lambda i: (0, 0)),
                pl.BlockSpec((1, E), lambda i: (0, 0)),
            ],
            out_specs=pl.BlockSpec((1, 1, TB), lambda i: (i, 0, 0)),
            out_shape=jax.ShapeDtypeStruct((nb, 1, TB), jnp.int32),
        )(x, W1, b1.reshape(1, H), W3, b3.reshape(1, E))

    idxq = run_router(query_emb)
    idxd = run_router(doc_emb)

    C = 128
    R = N // C
    dest2d, bexp2d = pl.pallas_call(
        functools.partial(_meta_body, E=E, T=T, NBLK=NBLK, R=R, C=C),
        grid=(1,),
        in_specs=[
            pl.BlockSpec((R // 2, C), lambda i: (0, 0)),
            pl.BlockSpec((R // 2, C), lambda i: (0, 0)),
        ],
        out_specs=[
            pl.BlockSpec((R, C), lambda i: (0, 0)),
            pl.BlockSpec((1, NBLK), lambda i: (0, 0)),
        ],
        out_shape=[
            jax.ShapeDtypeStruct((R, C), jnp.int32),
            jax.ShapeDtypeStruct((1, NBLK), jnp.int32),
        ],
    )(idxq.reshape(R // 2, C), idxd.reshape(R // 2, C))
    dest = dest2d.reshape(N)
    bexp = bexp2d.reshape(NBLK)
    destq = dest[:B].reshape(1, B)
    destd = dest[B:].reshape(1, B)

    S = 4
    DS = D // S
    mesh = plsc.VectorSubcoreMesh(core_axis_name="c", subcore_axis_name="s")
    W = 128

    @functools.partial(
        pl.kernel,
        mesh=mesh,
        out_type=[jax.ShapeDtypeStruct((NPAD, DS), f32) for _ in range(S)],
    )
    def sc_scatter(q_hbm, d_hbm, iq_hbm, id_hbm, xo0, xo1, xo2, xo3):
        outs = (xo0, xo1, xo2, xo3)
        for k in range(S):
            for src, idx in ((q_hbm, iq_hbm), (d_hbm, id_hbm)):
                def body(x_vmem, i_vmem, _out=outs[k]):
                    pltpu.sync_copy(x_vmem, _out.at[i_vmem.at[0]])

                pltpu.emit_pipeline(
                    body,
                    grid=(B // W,),
                    in_specs=[
                        pl.BlockSpec((W, DS), lambda i, _k=k: (i, _k)),
                        pl.BlockSpec((1, W), lambda i: (0, i)),
                    ],
                    out_specs=[],
                    core_axis_name=("c", "s"),
                    dimension_semantics=(pltpu.PARALLEL,),
                )(src, idx)

    xs_planes = sc_scatter(query_emb, doc_emb, destq, destd)

    grid_spec = pltpu.PrefetchScalarGridSpec(
        num_scalar_prefetch=1,
        grid=(NBLK,),
        in_specs=[
            pl.BlockSpec((T, DS), lambda i, be: (i, 0)),
            pl.BlockSpec((T, DS), lambda i, be: (i, 0)),
            pl.BlockSpec((T, DS), lambda i, be: (i, 0)),
            pl.BlockSpec((T, DS), lambda i, be: (i, 0)),
            pl.BlockSpec((1, S, DS, H), lambda i, be: (be[i], 0, 0, 0)),
            pl.BlockSpec((1, 1, H), lambda i, be: (be[i], 0, 0)),
            pl.BlockSpec((1, H, D), lambda i, be: (be[i], 0, 0)),
            pl.BlockSpec((1, 1, D), lambda i, be: (be[i], 0, 0)),
        ],
        out_specs=pl.BlockSpec((T, D), lambda i, be: (i, 0)),
        scratch_shapes=[
            pltpu.VMEM((S, DS, H), jnp.bfloat16),
            pltpu.VMEM((H, D), jnp.bfloat16),
        ],
    )
    res = pl.pallas_call(
        functools.partial(_expert_body, DS=DS),
        grid_spec=grid_spec,
        out_shape=jax.ShapeDtypeStruct((NPAD, D), f32),
    )(bexp, *xs_planes, Ws1.reshape(E, S, DS, H), bs1.reshape(E, 1, H),
      Ws2, bs2.reshape(E, 1, D))

    NW = 32
    RPW = N // NW
    CH = 32
    NCH = RPW // CH

    @functools.partial(
        pl.kernel,
        mesh=mesh,
        out_type=[
            jax.ShapeDtypeStruct((B, D), f32),
            jax.ShapeDtypeStruct((B, D), f32),
        ],
        scratch_types=[
            pltpu.VMEM((CH,), jnp.int32),
            pltpu.VMEM((CH,), jnp.int32),
            pltpu.VMEM((CH, D), f32),
            pltpu.VMEM((CH, D), f32),
            pltpu.SemaphoreType.DMA,
            pltpu.SemaphoreType.DMA,
            pltpu.SemaphoreType.DMA,
            pltpu.SemaphoreType.DMA,
        ],
    )
    def sc_gather(res_hbm, i_hbm, q_hbm, d_hbm,
                  idx_v0, idx_v1, rows_v0, rows_v1, sem0, sem1, wsem0, wsem1):
        wid = lax.axis_index("s") * 2 + lax.axis_index("c")
        base = wid * RPW
        idx_v = (idx_v0, idx_v1)
        rows_v = (rows_v0, rows_v1)
        sems = (sem0, sem1)
        wsems = (wsem0, wsem1)

        def out_copy(buf, j):
            row0 = base + j * CH

            @pl.when(wid < NW // 2)
            def _():
                pltpu.make_async_copy(
                    rows_v[buf], q_hbm.at[pl.ds(row0, CH)], wsems[buf]).start()

            @pl.when(wid >= NW // 2)
            def _():
                pltpu.make_async_copy(
                    rows_v[buf], d_hbm.at[pl.ds(row0 - B, CH)],
                    wsems[buf]).start()

        def out_wait(buf, j):
            row0 = base + j * CH

            @pl.when(wid < NW // 2)
            def _():
                pltpu.make_async_copy(
                    rows_v[buf], q_hbm.at[pl.ds(row0, CH)], wsems[buf]).wait()

            @pl.when(wid >= NW // 2)
            def _():
                pltpu.make_async_copy(
                    rows_v[buf], d_hbm.at[pl.ds(row0 - B, CH)],
                    wsems[buf]).wait()

        pltpu.sync_copy(i_hbm.at[pl.ds(base, CH)], idx_v[0])
        pltpu.make_async_copy(res_hbm.at[idx_v[0]], rows_v[0], sems[0]).start()
        for j in range(NCH):
            cur = j % 2
            nxt = 1 - cur
            if j + 1 < NCH:
                pltpu.sync_copy(
                    i_hbm.at[pl.ds(base + (j + 1) * CH, CH)], idx_v[nxt])
                if j - 1 >= 0:
                    out_wait(nxt, j - 1)
                pltpu.make_async_copy(
                    res_hbm.at[idx_v[nxt]], rows_v[nxt], sems[nxt]).start()
            pltpu.make_async_copy(
                res_hbm.at[idx_v[cur]], rows_v[cur], sems[cur]).wait()
            out_copy(cur, j)
        out_wait((NCH - 2) % 2, NCH - 2)
        out_wait((NCH - 1) % 2, NCH - 1)

    q_out, d_out = sc_gather(res, dest)
    return (q_out, d_out)


# speedup vs baseline: 1.0746x; 1.0448x over previous
import functools

import jax
import jax.numpy as jnp
from jax import lax
from jax.experimental import pallas as pl
from jax.experimental.pallas import tpu as pltpu
from jax.experimental.pallas import tpu_sc as plsc


def _router_body(x_ref, w1_ref, b1_ref, w3_ref, b3_ref, idx_ref):
    x = x_ref[...]
    h = jnp.maximum(
        jnp.dot(x, w1_ref[...], preferred_element_type=jnp.float32) + b1_ref[...],
        0.0,
    )
    logits = jnp.dot(h, w3_ref[...], preferred_element_type=jnp.float32) + b3_ref[...]
    idx_ref[0, 0, :] = jnp.argmax(logits, axis=-1).astype(jnp.int32)


def _meta_body(idxq_ref, idxd_ref, dest_ref, bexp_ref, *, E, T, NBLK, R, C):
    idx = jnp.concatenate([idxq_ref[...], idxd_ref[...]], axis=0)
    ohs = [idx == e for e in range(E)]
    counts = [jnp.sum(oh.astype(jnp.int32)) for oh in ohs]

    starts = []
    s = jnp.int32(0)
    for e in range(E):
        starts.append(s)
        s = s + ((counts[e] + (T - 1)) // T) * T

    cio = lax.broadcasted_iota(jnp.int32, (C, C), 0)
    cjo = lax.broadcasted_iota(jnp.int32, (C, C), 1)
    su = (cio < cjo).astype(jnp.float32)
    rio = lax.broadcasted_iota(jnp.int32, (R, R), 0)
    rjo = lax.broadcasted_iota(jnp.int32, (R, R), 1)
    sl = (rjo < rio).astype(jnp.float32)

    dest = jnp.zeros((R, C), jnp.int32)
    for e in range(E):
        ohf = ohs[e].astype(jnp.float32)
        within = jnp.dot(ohf, su, preferred_element_type=jnp.float32)
        rowsum = jnp.sum(ohf, axis=1, keepdims=True)
        rexcl = jnp.dot(sl, rowsum, preferred_element_type=jnp.float32)
        pos = (rexcl + within).astype(jnp.int32)
        dest = dest + jnp.where(ohs[e], starts[e] + pos, 0)
    dest_ref[...] = dest

    bio = lax.broadcasted_iota(jnp.int32, (1, NBLK), 1) * T
    be = jnp.zeros((1, NBLK), jnp.int32)
    for e in range(1, E):
        be = be + (bio >= starts[e]).astype(jnp.int32)
    bexp_ref[...] = be


def _expert_body(bexp_ref, x0_ref, x1_ref, x2_ref, x3_ref,
                 w1_ref, b1_ref, w2_ref, b2_ref, o_ref,
                 w1s_ref, w2s_ref, *, DS):
    i = pl.program_id(0)
    prev = bexp_ref[jnp.maximum(i - 1, 0)]

    @pl.when(jnp.logical_or(i == 0, bexp_ref[i] != prev))
    def _():
        w1s_ref[...] = w1_ref[0].astype(jnp.bfloat16)
        w2s_ref[...] = w2_ref[0].astype(jnp.bfloat16)

    xs = [x0_ref[...], x1_ref[...], x2_ref[...], x3_ref[...]]
    h = jnp.dot(xs[0].astype(jnp.bfloat16), w1s_ref[0],
                preferred_element_type=jnp.float32)
    for k in range(1, 4):
        h = h + jnp.dot(xs[k].astype(jnp.bfloat16), w1s_ref[k],
                        preferred_element_type=jnp.float32)
    h = jnp.maximum(h + b1_ref[0], 0.0)
    o = jnp.dot(h.astype(jnp.bfloat16), w2s_ref[...],
                preferred_element_type=jnp.float32) + b2_ref[0]
    inv = 1.0 / jnp.maximum(jnp.sqrt(jnp.sum(o * o, axis=1, keepdims=True)),
                            1e-6)
    y = o * inv
    for k in range(4):
        o_ref[:, k * DS:(k + 1) * DS] = y[:, k * DS:(k + 1) * DS] + xs[k]


def kernel(query_emb, doc_emb, W1, b1, W3, b3, Ws1, bs1, Ws2, bs2):
    B, D = query_emb.shape
    H = W1.shape[1]
    E = W3.shape[1]
    N = 2 * B
    T = 512
    NBLK = N // T + E
    NPAD = NBLK * T
    f32 = jnp.float32

    TB = 512
    nb = B // TB

    def run_router(x):
        return pl.pallas_call(
            _router_body,
            grid=(nb,),
            in_specs=[
                pl.BlockSpec((TB, D), lambda i: (i, 0)),
                pl.BlockSpec((D, H), lambda i: (0, 0)),
                pl.BlockSpec((1, H), lambda i: (0, 0)),
                pl.BlockSpec((H, E), lambda i: (0, 0)),
                pl.BlockSpec((1, E), lambda i: (0, 0)),
            ],
            out_specs=pl.BlockSpec((1, 1, TB), lambda i: (i, 0, 0)),
            out_shape=jax.ShapeDtypeStruct((nb, 1, TB), jnp.int32),
        )(x, W1, b1.reshape(1, H), W3, b3.reshape(1, E))

    idxq = run_router(query_emb)
    idxd = run_router(doc_emb)

    C = 128
    R = N // C
    dest2d, bexp2d = pl.pallas_call(
        functools.partial(_meta_body, E=E, T=T, NBLK=NBLK, R=R, C=C),
        grid=(1,),
        in_specs=[
            pl.BlockSpec((R // 2, C), lambda i: (0, 0)),
            pl.BlockSpec((R // 2, C), lambda i: (0, 0)),
        ],
        out_specs=[
            pl.BlockSpec((R, C), lambda i: (0, 0)),
            pl.BlockSpec((1, NBLK), lambda i: (0, 0)),
        ],
        out_shape=[
            jax.ShapeDtypeStruct((R, C), jnp.int32),
            jax.ShapeDtypeStruct((1, NBLK), jnp.int32),
        ],
    )(idxq.reshape(R // 2, C), idxd.reshape(R // 2, C))
    dest = dest2d.reshape(N)
    bexp = bexp2d.reshape(NBLK)
    destq = dest[:B].reshape(1, B)
    destd = dest[B:].reshape(1, B)

    S = 4
    DS = D // S
    mesh = plsc.VectorSubcoreMesh(core_axis_name="c", subcore_axis_name="s")
    W = 128

    @functools.partial(
        pl.kernel,
        mesh=mesh,
        out_type=[jax.ShapeDtypeStruct((NPAD, DS), f32) for _ in range(S)],
    )
    def sc_scatter(q_hbm, d_hbm, iq_hbm, id_hbm, xo0, xo1, xo2, xo3):
        outs = (xo0, xo1, xo2, xo3)
        for k in range(S):
            for src, idx in ((q_hbm, iq_hbm), (d_hbm, id_hbm)):
                def body(x_vmem, i_vmem, _out=outs[k]):
                    pltpu.sync_copy(x_vmem, _out.at[i_vmem.at[0]])

                pltpu.emit_pipeline(
                    body,
                    grid=(B // W,),
                    in_specs=[
                        pl.BlockSpec((W, DS), lambda i, _k=k: (i, _k)),
                        pl.BlockSpec((1, W), lambda i: (0, i)),
                    ],
                    out_specs=[],
                    core_axis_name=("c", "s"),
                    dimension_semantics=(pltpu.PARALLEL,),
                )(src, idx)

    xs_planes = sc_scatter(query_emb, doc_emb, destq, destd)

    grid_spec = pltpu.PrefetchScalarGridSpec(
        num_scalar_prefetch=1,
        grid=(NBLK,),
        in_specs=[
            pl.BlockSpec((T, DS), lambda i, be: (i, 0)),
            pl.BlockSpec((T, DS), lambda i, be: (i, 0)),
            pl.BlockSpec((T, DS), lambda i, be: (i, 0)),
            pl.BlockSpec((T, DS), lambda i, be: (i, 0)),
            pl.BlockSpec((1, S, DS, H), lambda i, be: (be[i], 0, 0, 0)),
            pl.BlockSpec((1, 1, H), lambda i, be: (be[i], 0, 0)),
            pl.BlockSpec((1, H, D), lambda i, be: (be[i], 0, 0)),
            pl.BlockSpec((1, 1, D), lambda i, be: (be[i], 0, 0)),
        ],
        out_specs=pl.BlockSpec((T, D), lambda i, be: (i, 0)),
        scratch_shapes=[
            pltpu.VMEM((S, DS, H), jnp.bfloat16),
            pltpu.VMEM((H, D), jnp.bfloat16),
        ],
    )
    res = pl.pallas_call(
        functools.partial(_expert_body, DS=DS),
        grid_spec=grid_spec,
        out_shape=jax.ShapeDtypeStruct((NPAD, D), f32),
    )(bexp, *xs_planes, Ws1.reshape(E, S, DS, H), bs1.reshape(E, 1, H),
      Ws2, bs2.reshape(E, 1, D))

    NW = 32
    RPW = N // NW
    CH = 32
    NCH = RPW // CH

    @functools.partial(
        pl.kernel,
        mesh=mesh,
        out_type=[
            jax.ShapeDtypeStruct((B, D), f32),
            jax.ShapeDtypeStruct((B, D), f32),
        ],
        scratch_types=[
            pltpu.VMEM((CH,), jnp.int32),
            pltpu.VMEM((CH,), jnp.int32),
            pltpu.VMEM((CH, D), f32),
            pltpu.VMEM((CH, D), f32),
            pltpu.SemaphoreType.DMA,
            pltpu.SemaphoreType.DMA,
            pltpu.SemaphoreType.DMA,
            pltpu.SemaphoreType.DMA,
        ],
    )
    def sc_gather(res_hbm, i_hbm, q_hbm, d_hbm,
                  idx_v0, idx_v1, rows_v0, rows_v1, sem0, sem1, wsem0, wsem1):
        wid = lax.axis_index("s") * 2 + lax.axis_index("c")
        base = wid * RPW
        idx_v = (idx_v0, idx_v1)
        rows_v = (rows_v0, rows_v1)
        sems = (sem0, sem1)
        wsems = (wsem0, wsem1)

        def out_copy(buf, j):
            row0 = base + j * CH

            @pl.when(wid < NW // 2)
            def _():
                pltpu.make_async_copy(
                    rows_v[buf], q_hbm.at[pl.ds(row0, CH)], wsems[buf]).start()

            @pl.when(wid >= NW // 2)
            def _():
                pltpu.make_async_copy(
                    rows_v[buf], d_hbm.at[pl.ds(row0 - B, CH)],
                    wsems[buf]).start()

        def out_wait(buf, j):
            row0 = base + j * CH

            @pl.when(wid < NW // 2)
            def _():
                pltpu.make_async_copy(
                    rows_v[buf], q_hbm.at[pl.ds(row0, CH)], wsems[buf]).wait()

            @pl.when(wid >= NW // 2)
            def _():
                pltpu.make_async_copy(
                    rows_v[buf], d_hbm.at[pl.ds(row0 - B, CH)],
                    wsems[buf]).wait()

        pltpu.sync_copy(i_hbm.at[pl.ds(base, CH)], idx_v[0])
        pltpu.make_async_copy(res_hbm.at[idx_v[0]], rows_v[0], sems[0]).start()
        for j in range(NCH):
            cur = j % 2
            nxt = 1 - cur
            if j + 1 < NCH:
                pltpu.sync_copy(
                    i_hbm.at[pl.ds(base + (j + 1) * CH, CH)], idx_v[nxt])
                if j - 1 >= 0:
                    out_wait(nxt, j - 1)
                pltpu.make_async_copy(
                    res_hbm.at[idx_v[nxt]], rows_v[nxt], sems[nxt]).start()
            pltpu.make_async_copy(
                res_hbm.at[idx_v[cur]], rows_v[cur], sems[cur]).wait()
            out_copy(cur, j)
        out_wait((NCH - 2) % 2, NCH - 2)
        out_wait((NCH - 1) % 2, NCH - 1)

    q_out, d_out = sc_gather(res, dest)
    return (q_out, d_out)
